# Initial kernel scaffold; baseline (speedup 1.0000x reference)
import jax
import jax.numpy as jnp
from jax.experimental import pallas as pl


def kernel(embeddings, input1, input2):
    raise NotImplementedError("write your pallas kernel here")


# speedup vs baseline: 1.8874x; 1.8874x over previous
import functools

import jax
import jax.numpy as jnp
from jax import lax
from jax.experimental import pallas as pl
from jax.experimental.pallas import tpu as pltpu
from jax.experimental.pallas import tpu_sc as plsc


@functools.cache
def _make_lookup(BL: int, E: int, C: int):
    info = plsc.get_sparse_core_info()
    NC, NS = info.num_cores, info.num_subcores
    NW = NC * NS
    per_w = BL // NW
    n_chunks = per_w // C
    assert per_w % C == 0 and BL % NW == 0

    mesh = plsc.VectorSubcoreMesh(core_axis_name="c", subcore_axis_name="s")

    @functools.partial(
        pl.kernel,
        mesh=mesh,
        compiler_params=pltpu.CompilerParams(use_tc_tiling_on_sc=False),
        out_type=jax.ShapeDtypeStruct((2, BL, E), jnp.float32),
        scratch_types=[
            pltpu.VMEM((C,), jnp.int32),
            pltpu.VMEM((C, E), jnp.float32),
            pltpu.SemaphoreType.DMA,
        ],
    )
    def lookup(table, idx1, idx2, out, idx_v, rows_v, sem):
        wid = lax.axis_index("s") * NC + lax.axis_index("c")
        base = wid * per_w

        def body(i, _):
            off = base + i * C
            pltpu.sync_copy(idx1.at[pl.ds(off, C)], idx_v)
            pltpu.async_copy(table.at[idx_v], rows_v, sem).wait()
            pltpu.sync_copy(rows_v, out.at[0, pl.ds(off, C)])
            pltpu.sync_copy(idx2.at[pl.ds(off, C)], idx_v)
            pltpu.async_copy(table.at[idx_v], rows_v, sem).wait()
            pltpu.sync_copy(rows_v, out.at[1, pl.ds(off, C)])
            return ()

        lax.fori_loop(0, n_chunks, body, ())

    return lookup


def kernel(embeddings, input1, input2):
    b, l, nf = input1.shape
    e = embeddings.shape[1]
    BL = b * l * nf
    idx1 = input1.reshape(BL)
    idx2 = input2.reshape(BL)
    out = _make_lookup(BL, e, 1024)(embeddings, idx1, idx2)
    return out.reshape(2, b, l, nf * e)


# speedup vs baseline: 1.9037x; 1.0087x over previous
import functools

import jax
import jax.numpy as jnp
from jax import lax
from jax.experimental import pallas as pl
from jax.experimental.pallas import tpu as pltpu
from jax.experimental.pallas import tpu_sc as plsc


@functools.cache
def _make_lookup(B: int, L: int, E: int, NB: int):
    info = plsc.get_sparse_core_info()
    NC, NS = info.num_cores, info.num_subcores
    NW = NC * NS
    b_per_w = B // NW
    n_chunks = b_per_w // NB
    C = NB * L
    assert B % NW == 0 and b_per_w % NB == 0

    mesh = plsc.VectorSubcoreMesh(core_axis_name="c", subcore_axis_name="s")

    @functools.partial(
        pl.kernel,
        mesh=mesh,
        compiler_params=pltpu.CompilerParams(use_tc_tiling_on_sc=False),
        out_type=jax.ShapeDtypeStruct((2, B, L, E), jnp.float32),
        scratch_types=[
            pltpu.VMEM((C,), jnp.int32),
            pltpu.VMEM((C, E), jnp.float32),
            pltpu.SemaphoreType.DMA,
        ],
    )
    def lookup(table, idx1, idx2, out, idx_v, rows_v, sem):
        wid = lax.axis_index("s") * NC + lax.axis_index("c")
        base_b = wid * b_per_w

        def body(i, _):
            b0 = base_b + i * NB
            off = b0 * L
            for s, idx in ((0, idx1), (1, idx2)):
                pltpu.sync_copy(idx.at[pl.ds(off, C)], idx_v)
                pltpu.async_copy(table.at[idx_v], rows_v, sem).wait()
                for j in range(NB):
                    pltpu.sync_copy(
                        rows_v.at[pl.ds(j * L, L)], out.at[s, b0 + j]
                    )
            return ()

        lax.fori_loop(0, n_chunks, body, ())

    return lookup


def kernel(embeddings, input1, input2):
    b, l, nf = input1.shape
    e = embeddings.shape[1]
    idx1 = input1.reshape(b * l * nf)
    idx2 = input2.reshape(b * l * nf)
    return _make_lookup(b, l * nf, e, 8)(embeddings, idx1, idx2)
